# baseline (speedup 1.0000x reference)
import dataclasses
import functools

import jax
import jax.numpy as jnp
from jax import lax
from jax.experimental import pallas as pl
from jax.experimental.pallas import tpu as pltpu
from jax.experimental.pallas import tpu_sc as plsc

_N = 10000
_E = 320000
_D = 128
_DOUT = 200
_WIN = 128
_NWIN = _E // _WIN
_NC = 2
_NS = 16
_NW = _NC * _NS
_GM = 78
_NEPI = _NWIN - _GM * _NW
_NROW = 3
_NPK = 4
_UNR = 12
_RPT = 624
_RREM = _N - _RPT * _NS


def _sc_agg(x, src2, dst2, w2):
    mesh = plsc.VectorSubcoreMesh(core_axis_name="c", subcore_axis_name="s")
    cp = pltpu.CompilerParams()
    if "needs_layout_passes" in pltpu.CompilerParams.__dataclass_fields__:
        cp = dataclasses.replace(cp, needs_layout_passes=False)

    @functools.partial(
        pl.kernel,
        compiler_params=cp,
        out_type=jax.ShapeDtypeStruct((_NC, _N, _D), jnp.float32),
        mesh=mesh,
        scratch_types=[
            pltpu.VMEM((_NROW, _WIN, _D), jnp.float32),
            pltpu.VMEM((_NPK, 1, _WIN), jnp.int32),
            pltpu.VMEM((_NPK, 1, _WIN), jnp.int32),
            pltpu.VMEM((_NPK, 1, _WIN), jnp.float32),
            pltpu.VMEM_SHARED((_N, _D), jnp.float32),
            pltpu.SemaphoreType.DMA((_NROW,)),
            pltpu.SemaphoreType.DMA((_NROW,)),
            pltpu.SemaphoreType.DMA((_NPK,)),
        ],
    )
    def k(x_hbm, s_hbm, d_hbm, w_hbm, out_hbm,
          rows, sbuf, dbuf, wbuf, acc, gsem, ssem, pksem):
        c = lax.axis_index("c")
        s = lax.axis_index("s")
        wid = c * _NS + s
        zero16 = jnp.zeros((16,), jnp.int32)

        def win_of(g):
            return g * _NW + wid

        def pk_start(g, p):
            w = pl.ds(win_of(g), 1)
            pltpu.async_copy(s_hbm.at[w], sbuf.at[p], pksem.at[p])
            pltpu.async_copy(d_hbm.at[w], dbuf.at[p], pksem.at[p])
            pltpu.async_copy(w_hbm.at[w], wbuf.at[p], pksem.at[p])

        def pk_wait(g, p):
            w = pl.ds(win_of(g), 1)
            pltpu.make_async_copy(s_hbm.at[w], sbuf.at[p], pksem.at[p]).wait()
            pltpu.make_async_copy(d_hbm.at[w], dbuf.at[p], pksem.at[p]).wait()
            pltpu.make_async_copy(w_hbm.at[w], wbuf.at[p], pksem.at[p]).wait()

        def gather_start(p, b):
            pltpu.async_copy(x_hbm.at[sbuf.at[p, 0]], rows.at[b], gsem.at[b])

        def gather_wait(p, b):
            pltpu.make_async_copy(x_hbm.at[sbuf.at[p, 0]], rows.at[b],
                                  gsem.at[b]).wait()

        def scat_start(p, b):
            pltpu.async_copy(rows.at[b], acc.at[dbuf.at[p, 0]],
                             ssem.at[b], add=True)

        def scat_wait(p, b):
            pltpu.make_async_copy(rows.at[b], acc.at[dbuf.at[p, 0]],
                                  ssem.at[b]).wait()

        def multiply(b, p):
            r = rows.at[b]

            @plsc.parallel_loop(0, _WIN, unroll=4)
            def _(e):
                wv = plsc.load_gather(wbuf, [zero16 + p, zero16, zero16 + e])
                for j in range(_D // 16):
                    sl = pl.ds(j * 16, 16)
                    r[e, sl] = r[e, sl] * wv

        for g in range(_NPK - 1):
            pk_start(g, g)

        @pl.loop(0, _WIN)
        def _(i):
            for j in range(_D // 16):
                rows[0, i, pl.ds(j * 16, 16)] = jnp.zeros((16,), jnp.float32)

        r0 = s * _RPT
        zsrc = rows.at[0]
        _zsems = [gsem.at[0], gsem.at[1], gsem.at[2], ssem.at[0], ssem.at[1]]
        _zcps = []
        for t in range(_RPT // _WIN):
            _zcps.append(pltpu.async_copy(
                zsrc, acc.at[pl.ds(r0 + t * _WIN, _WIN)], _zsems[t]))
        _zr = _RPT % _WIN
        if _zr:
            _zcps.append(pltpu.async_copy(
                zsrc.at[pl.ds(0, _zr)],
                acc.at[pl.ds(r0 + (_RPT // _WIN) * _WIN, _zr)],
                _zsems[_RPT // _WIN]))

        @pl.when(s == _NS - 1)
        def _():
            pltpu.async_copy(zsrc.at[pl.ds(0, _RREM)],
                             acc.at[pl.ds(_RPT * _NS, _RREM)],
                             ssem.at[2]).wait()
        for _cp in _zcps:
            _cp.wait()

        plsc.subcore_barrier()

        pk_wait(0, 0)
        gather_start(0, 0)
        pk_wait(1, 1)
        gather_start(1, 1)
        gather_wait(0, 0)
        multiply(0, 0)
        pk_start(_NPK - 1, _NPK - 1)
        pk_wait(2, 2)
        gather_start(2, 2)
        scat_start(0, 0)

        @pl.loop(0, -(-(_GM - 1) // _UNR))
        def _(q):
            for k12 in range(_UNR):
                g = 1 + q * _UNR + k12
                b = (1 + k12) % _NROW
                bp = k12 % _NROW
                p = (1 + k12) % _NPK
                pp = k12 % _NPK
                p2 = (3 + k12) % _NPK
                p3 = (4 + k12) % _NPK

                @pl.when(g < _GM)
                def _():
                    gather_wait(p, b)
                    scat_wait(pp, bp)

                    @pl.when(g + 2 < _GM)
                    def _():
                        pk_wait(g + 2, p2)
                        gather_start(p2, (3 + k12) % _NROW)

                    multiply(b, p)
                    scat_start(p, b)

                    @pl.when(g + 3 < _GM)
                    def _():
                        pk_start(g + 3, p3)

        scat_wait((_GM - 1) % _NPK, (_GM - 1) % _NROW)

        @pl.when(wid < _NEPI)
        def _():
            w = pl.ds(_GM * _NW + wid, 1)
            pltpu.sync_copy(s_hbm.at[w], sbuf.at[0])
            pltpu.sync_copy(d_hbm.at[w], dbuf.at[0])
            pltpu.sync_copy(w_hbm.at[w], wbuf.at[0])
            pltpu.async_copy(x_hbm.at[sbuf.at[0, 0]], rows.at[0],
                             gsem.at[0]).wait()
            multiply(0, 0)
            pltpu.sync_copy(rows.at[0], acc.at[dbuf.at[0, 0]], add=True)

        plsc.subcore_barrier()
        pltpu.sync_copy(acc.at[pl.ds(r0, _RPT)],
                        out_hbm.at[c, pl.ds(r0, _RPT)])

        @pl.when(s == _NS - 1)
        def _():
            pltpu.sync_copy(acc.at[pl.ds(_RPT * _NS, _RREM)],
                            out_hbm.at[c, pl.ds(_RPT * _NS, _RREM)])

    return k(x, src2, dst2, w2)


def _tc_root(x, W_root, b_rel):
    BN = 1000

    def body(x_ref, wro_ref, br_ref, o_ref):
        o_ref[...] = (
            jnp.dot(x_ref[...], wro_ref[...], preferred_element_type=jnp.float32)
            + br_ref[...]
        )

    full = lambda shape: pl.BlockSpec(shape, lambda i: (0,) * len(shape))
    return pl.pallas_call(
        body,
        grid=(_N // BN,),
        in_specs=[pl.BlockSpec((BN, _D), lambda i: (i, 0)),
                  full((_D, _DOUT)), full((1, _DOUT))],
        out_specs=pl.BlockSpec((BN, _DOUT), lambda i: (i, 0)),
        out_shape=jax.ShapeDtypeStruct((_N, _DOUT), jnp.float32),
    )(x, W_root, b_rel)


def _tc_post(p, xr, W_rel, W_dec, b_dec):
    BN = 1000

    def body(p0_ref, p1_ref, xr_ref, wr_ref, wd_ref, bd_ref, o_ref):
        agg = p0_ref[0] + p1_ref[0]
        z = jnp.dot(agg, wr_ref[...], preferred_element_type=jnp.float32)
        z = jnp.maximum(z + xr_ref[...], 0.0)
        o_ref[...] = (
            jnp.dot(z, wd_ref[...], preferred_element_type=jnp.float32) + bd_ref[...]
        )

    full = lambda shape: pl.BlockSpec(shape, lambda i: (0,) * len(shape))
    return pl.pallas_call(
        body,
        grid=(_N // BN,),
        in_specs=[
            pl.BlockSpec((1, BN, _D), lambda i: (0, i, 0)),
            pl.BlockSpec((1, BN, _D), lambda i: (1, i, 0)),
            pl.BlockSpec((BN, _DOUT), lambda i: (i, 0)),
            full((_D, _DOUT)), full((_DOUT, _D)), full((1, _D)),
        ],
        out_specs=pl.BlockSpec((BN, _D), lambda i: (i, 0)),
        out_shape=jax.ShapeDtypeStruct((_N, _D), jnp.float32),
    )(p, p, xr, W_rel, W_dec, b_dec)


def kernel(x, edge_index, edge_weight, W_rel, b_rel, W_root, W_dec, b_dec):
    ei = edge_index.astype(jnp.int32)
    src2 = ei[0].reshape(_NWIN, _WIN)
    dst2 = ei[1].reshape(_NWIN, _WIN)
    w2 = edge_weight.reshape(_NWIN, _WIN)
    xr = _tc_root(x, W_root, b_rel.reshape(1, _DOUT))
    p = _sc_agg(x, src2, dst2, w2)
    return _tc_post(p, xr, W_rel, W_dec, b_dec.reshape(1, _D))


# speedup vs baseline: 1.0349x; 1.0349x over previous
import dataclasses
import functools

import jax
import jax.numpy as jnp
from jax import lax
from jax.experimental import pallas as pl
from jax.experimental.pallas import tpu as pltpu
from jax.experimental.pallas import tpu_sc as plsc

_N = 10000
_E = 320000
_D = 128
_DOUT = 200
_WIN = 128
_NWIN = _E // _WIN
_NC = 2
_NS = 16
_NW = _NC * _NS
_GM = 78
_NEPI = _NWIN - _GM * _NW
_NROW = 3
_NPK = 4
_UNR = 12
_RPT = 624
_RREM = _N - _RPT * _NS


def _sc_agg(x, src2, dst2, w2):
    mesh = plsc.VectorSubcoreMesh(core_axis_name="c", subcore_axis_name="s")
    cp = pltpu.CompilerParams()
    if "needs_layout_passes" in pltpu.CompilerParams.__dataclass_fields__:
        cp = dataclasses.replace(cp, needs_layout_passes=False)

    @functools.partial(
        pl.kernel,
        compiler_params=cp,
        out_type=jax.ShapeDtypeStruct((_NC, _N, _D), jnp.float32),
        mesh=mesh,
        scratch_types=[
            pltpu.VMEM((_NROW, _WIN, _D), jnp.float32),
            pltpu.VMEM((_NPK, 1, _WIN), jnp.int32),
            pltpu.VMEM((_NPK, 1, _WIN), jnp.int32),
            pltpu.VMEM((_NPK, 1, _WIN), jnp.float32),
            pltpu.VMEM_SHARED((_N, _D), jnp.float32),
            pltpu.SemaphoreType.DMA((_NROW,)),
            pltpu.SemaphoreType.DMA((_NROW,)),
            pltpu.SemaphoreType.DMA((_NPK,)),
        ],
    )
    def k(x_hbm, s_hbm, d_hbm, w_hbm, out_hbm,
          rows, sbuf, dbuf, wbuf, acc, gsem, ssem, pksem):
        c = lax.axis_index("c")
        s = lax.axis_index("s")
        wid = c * _NS + s
        zero16 = jnp.zeros((16,), jnp.int32)

        def win_of(g):
            return g * _NW + wid

        def pk_start(g, p):
            w = pl.ds(win_of(g), 1)
            pltpu.async_copy(s_hbm.at[w], sbuf.at[p], pksem.at[p])
            pltpu.async_copy(d_hbm.at[w], dbuf.at[p], pksem.at[p])
            pltpu.async_copy(w_hbm.at[w], wbuf.at[p], pksem.at[p])

        def pk_wait(g, p):
            w = pl.ds(win_of(g), 1)
            pltpu.make_async_copy(s_hbm.at[w], sbuf.at[p], pksem.at[p]).wait()
            pltpu.make_async_copy(d_hbm.at[w], dbuf.at[p], pksem.at[p]).wait()
            pltpu.make_async_copy(w_hbm.at[w], wbuf.at[p], pksem.at[p]).wait()

        def gather_start(p, b):
            pltpu.async_copy(x_hbm.at[sbuf.at[p, 0]], rows.at[b], gsem.at[b])

        def gather_wait(p, b):
            pltpu.make_async_copy(x_hbm.at[sbuf.at[p, 0]], rows.at[b],
                                  gsem.at[b]).wait()

        def scat_start(p, b):
            pltpu.async_copy(rows.at[b], acc.at[dbuf.at[p, 0]],
                             ssem.at[b], add=True)

        def scat_wait(p, b):
            pltpu.make_async_copy(rows.at[b], acc.at[dbuf.at[p, 0]],
                                  ssem.at[b]).wait()

        def multiply(b, p):
            r = rows.at[b]

            @plsc.parallel_loop(0, _WIN, unroll=4)
            def _(e):
                wv = plsc.load_gather(wbuf, [zero16 + p, zero16, zero16 + e])
                for j in range(_D // 16):
                    sl = pl.ds(j * 16, 16)
                    r[e, sl] = r[e, sl] * wv

        for g in range(_NPK - 1):
            pk_start(g, g)

        @pl.loop(0, _WIN)
        def _(i):
            for j in range(_D // 16):
                rows[0, i, pl.ds(j * 16, 16)] = jnp.zeros((16,), jnp.float32)

        r0 = s * _RPT
        zsrc = rows.at[0]
        _zsems = [gsem.at[0], gsem.at[1], gsem.at[2], ssem.at[0], ssem.at[1]]
        _zcps = []
        for t in range(_RPT // _WIN):
            _zcps.append(pltpu.async_copy(
                zsrc, acc.at[pl.ds(r0 + t * _WIN, _WIN)], _zsems[t]))
        _zr = _RPT % _WIN
        if _zr:
            _zcps.append(pltpu.async_copy(
                zsrc.at[pl.ds(0, _zr)],
                acc.at[pl.ds(r0 + (_RPT // _WIN) * _WIN, _zr)],
                _zsems[_RPT // _WIN]))

        @pl.when(s == _NS - 1)
        def _():
            pltpu.async_copy(zsrc.at[pl.ds(0, _RREM)],
                             acc.at[pl.ds(_RPT * _NS, _RREM)],
                             ssem.at[2]).wait()
        for _cp in _zcps:
            _cp.wait()

        plsc.subcore_barrier()

        pk_wait(0, 0)
        gather_start(0, 0)
        pk_wait(1, 1)
        gather_start(1, 1)
        gather_wait(0, 0)
        multiply(0, 0)
        pk_start(_NPK - 1, _NPK - 1)
        pk_wait(2, 2)
        gather_start(2, 2)
        scat_start(0, 0)

        @pl.loop(0, -(-(_GM - 1) // _UNR))
        def _(q):
            for k12 in range(_UNR):
                g = 1 + q * _UNR + k12
                b = (1 + k12) % _NROW
                bp = k12 % _NROW
                p = (1 + k12) % _NPK
                pp = k12 % _NPK
                p2 = (3 + k12) % _NPK
                p3 = (4 + k12) % _NPK

                @pl.when(g < _GM)
                def _():
                    gather_wait(p, b)
                    multiply(b, p)
                    scat_wait(pp, bp)

                    @pl.when(g + 3 < _GM)
                    def _():
                        pk_start(g + 3, p3)

                    @pl.when(g + 2 < _GM)
                    def _():
                        pk_wait(g + 2, p2)
                        gather_start(p2, (3 + k12) % _NROW)

                    scat_start(p, b)

        scat_wait((_GM - 1) % _NPK, (_GM - 1) % _NROW)

        @pl.when(wid < _NEPI)
        def _():
            w = pl.ds(_GM * _NW + wid, 1)
            pltpu.sync_copy(s_hbm.at[w], sbuf.at[0])
            pltpu.sync_copy(d_hbm.at[w], dbuf.at[0])
            pltpu.sync_copy(w_hbm.at[w], wbuf.at[0])
            pltpu.async_copy(x_hbm.at[sbuf.at[0, 0]], rows.at[0],
                             gsem.at[0]).wait()
            multiply(0, 0)
            pltpu.sync_copy(rows.at[0], acc.at[dbuf.at[0, 0]], add=True)

        plsc.subcore_barrier()
        pltpu.sync_copy(acc.at[pl.ds(r0, _RPT)],
                        out_hbm.at[c, pl.ds(r0, _RPT)])

        @pl.when(s == _NS - 1)
        def _():
            pltpu.sync_copy(acc.at[pl.ds(_RPT * _NS, _RREM)],
                            out_hbm.at[c, pl.ds(_RPT * _NS, _RREM)])

    return k(x, src2, dst2, w2)


def _tc_root(x, W_root, b_rel):
    BN = 1000

    def body(x_ref, wro_ref, br_ref, o_ref):
        o_ref[...] = (
            jnp.dot(x_ref[...], wro_ref[...], preferred_element_type=jnp.float32)
            + br_ref[...]
        )

    full = lambda shape: pl.BlockSpec(shape, lambda i: (0,) * len(shape))
    return pl.pallas_call(
        body,
        grid=(_N // BN,),
        in_specs=[pl.BlockSpec((BN, _D), lambda i: (i, 0)),
                  full((_D, _DOUT)), full((1, _DOUT))],
        out_specs=pl.BlockSpec((BN, _DOUT), lambda i: (i, 0)),
        out_shape=jax.ShapeDtypeStruct((_N, _DOUT), jnp.float32),
    )(x, W_root, b_rel)


def _tc_post(p, xr, W_rel, W_dec, b_dec):
    BN = 1000

    def body(p0_ref, p1_ref, xr_ref, wr_ref, wd_ref, bd_ref, o_ref):
        agg = p0_ref[0] + p1_ref[0]
        z = jnp.dot(agg, wr_ref[...], preferred_element_type=jnp.float32)
        z = jnp.maximum(z + xr_ref[...], 0.0)
        o_ref[...] = (
            jnp.dot(z, wd_ref[...], preferred_element_type=jnp.float32) + bd_ref[...]
        )

    full = lambda shape: pl.BlockSpec(shape, lambda i: (0,) * len(shape))
    return pl.pallas_call(
        body,
        grid=(_N // BN,),
        in_specs=[
            pl.BlockSpec((1, BN, _D), lambda i: (0, i, 0)),
            pl.BlockSpec((1, BN, _D), lambda i: (1, i, 0)),
            pl.BlockSpec((BN, _DOUT), lambda i: (i, 0)),
            full((_D, _DOUT)), full((_DOUT, _D)), full((1, _D)),
        ],
        out_specs=pl.BlockSpec((BN, _D), lambda i: (i, 0)),
        out_shape=jax.ShapeDtypeStruct((_N, _D), jnp.float32),
    )(p, p, xr, W_rel, W_dec, b_dec)


def kernel(x, edge_index, edge_weight, W_rel, b_rel, W_root, W_dec, b_dec):
    ei = edge_index.astype(jnp.int32)
    src2 = ei[0].reshape(_NWIN, _WIN)
    dst2 = ei[1].reshape(_NWIN, _WIN)
    w2 = edge_weight.reshape(_NWIN, _WIN)
    xr = _tc_root(x, W_root, b_rel.reshape(1, _DOUT))
    p = _sc_agg(x, src2, dst2, w2)
    return _tc_post(p, xr, W_rel, W_dec, b_dec.reshape(1, _D))
